# baseline (speedup 1.0000x reference)
import functools

import jax
import jax.numpy as jnp
from jax import lax
from jax.experimental import pallas as pl
from jax.experimental.pallas import tpu as pltpu
from jax.experimental.pallas import tpu_sc as plsc

VOCAB = 1000000
EMBED = 64
NUM_CLASS = 16
B = 16384

BLK = 2048
N_BLOCKS = B // BLK

NBUF = 8


def _sc_gather_cols(tableT, text):
    info = plsc.get_sparse_core_info()
    nc, ns = info.num_cores, info.num_subcores
    nw = nc * ns
    b_per_w = B // nw
    n_outer = b_per_w // NBUF

    mesh = plsc.VectorSubcoreMesh(core_axis_name="c", subcore_axis_name="s")

    @functools.partial(
        pl.kernel,
        mesh=mesh,
        out_type=jax.ShapeDtypeStruct((EMBED, B), jnp.float32),
        compiler_params=pltpu.CompilerParams(needs_layout_passes=False),
        scratch_types=[
            pltpu.VMEM((NBUF, EMBED, 128), jnp.float32),
            pltpu.VMEM((EMBED, b_per_w), jnp.float32),
            pltpu.VMEM((b_per_w,), jnp.int32),
        ] + [pltpu.SemaphoreType.DMA] * NBUF,
    )
    def gather_k(tableT_hbm, text_hbm, out_hbm, tiles_v, stage_v, idx_v,
                 *fsems):
        wid = lax.axis_index("s") * nc + lax.axis_index("c")
        base = wid * b_per_w

        pltpu.sync_copy(text_hbm.at[pl.ds(base, b_per_w)], idx_v)

        def get_idx(gbase, c):
            vec = idx_v[pl.ds(gbase, 16)]
            sel = jnp.where(lax.iota(jnp.int32, 16) == c, vec, 0)
            return jnp.sum(sel)

        def fetch(i, slot):
            off = pl.multiple_of((i >> 7) << 7, 128)
            pltpu.async_copy(
                tableT_hbm.at[:, pl.ds(off, 128)],
                tiles_v.at[slot],
                fsems[slot],
            )

        for s in range(NBUF):
            fetch(get_idx(0, s), s)

        def outer(o, carry):
            gbase = (o >> 1) * 16
            half = (o & 1) * NBUF
            for s in range(NBUF):
                pltpu.make_async_copy(
                    tableT_hbm.at[:, pl.ds(0, 128)],
                    tiles_v.at[s],
                    fsems[s],
                ).wait()
                i = get_idx(gbase, half + s)
                lane = jnp.full((16,), i & 127, jnp.int32)
                col = jnp.full((16,), o * NBUF + s, jnp.int32)
                for k in range(EMBED // 16):
                    rows = lax.iota(jnp.int32, 16) + (16 * k)
                    vals = plsc.load_gather(tiles_v.at[s], [rows, lane])
                    plsc.store_scatter(stage_v, [rows, col], vals)
                @pl.when(o < n_outer - 1)
                def _refill():
                    fetch(get_idx(((o + 1) >> 1) * 16, ((o + 1) & 1) * NBUF + s),
                          s)

            return carry

        lax.fori_loop(0, n_outer, outer, 0, unroll=False)

        pltpu.sync_copy(stage_v, out_hbm.at[:, pl.ds(base, b_per_w)])

    return gather_k(tableT, text)


def _mlp_loss_body(emb_ref, lab_ref, w1_ref, b1_ref, w2_ref, b2_ref,
                   wfc_ref, bfc_ref, out_ref):
    x = emb_ref[...]
    cdims = (((0,), (0,)), ((), ()))
    h = jnp.maximum(
        lax.dot_general(w1_ref[...], x, cdims,
                        preferred_element_type=jnp.float32) + b1_ref[...], 0.0)
    h = jnp.maximum(
        lax.dot_general(w2_ref[...], h, cdims,
                        preferred_element_type=jnp.float32) + b2_ref[...], 0.0)
    logits = lax.dot_general(wfc_ref[...], h, cdims,
                             preferred_element_type=jnp.float32) + bfc_ref[...]
    m = jnp.max(logits, axis=0, keepdims=True)
    lse = jnp.log(jnp.sum(jnp.exp(logits - m), axis=0, keepdims=True)) + m
    lab = lab_ref[...].reshape(1, BLK)
    cls = lax.broadcasted_iota(jnp.int32, (NUM_CLASS, BLK), 0)
    picked = jnp.sum(jnp.where(cls == lab, logits, 0.0),
                     axis=0, keepdims=True)
    part = jnp.sum(lse - picked, axis=(0, 1), keepdims=True)

    @pl.when(pl.program_id(0) == 0)
    def _init():
        out_ref[...] = jnp.zeros_like(out_ref)

    out_ref[...] += part


def _mlp_loss(embT, labels3d, W1, b1, W2, b2, Wfc, bfc):
    return pl.pallas_call(
        _mlp_loss_body,
        grid=(N_BLOCKS,),
        in_specs=[
            pl.BlockSpec((EMBED, BLK), lambda i: (0, i)),
            pl.BlockSpec((1, 1, BLK), lambda i: (i, 0, 0)),
            pl.BlockSpec((EMBED, EMBED), lambda i: (0, 0)),
            pl.BlockSpec((EMBED, 1), lambda i: (0, 0)),
            pl.BlockSpec((EMBED, EMBED), lambda i: (0, 0)),
            pl.BlockSpec((EMBED, 1), lambda i: (0, 0)),
            pl.BlockSpec((EMBED, NUM_CLASS), lambda i: (0, 0)),
            pl.BlockSpec((NUM_CLASS, 1), lambda i: (0, 0)),
        ],
        out_specs=pl.BlockSpec((1, 1), lambda i: (0, 0)),
        out_shape=jax.ShapeDtypeStruct((1, 1), jnp.float32),
    )(embT, labels3d, W1, b1, W2, b2, Wfc, bfc)


def kernel(text, offsets, labels, table, W1, b1, W2, b2, Wfc, bfc):
    text_s, labels_s = lax.sort(
        (text.astype(jnp.int32), labels.astype(jnp.int32)), num_keys=1)
    embT = _sc_gather_cols(table.T, text_s)
    loss_sum = _mlp_loss(
        embT,
        labels_s.reshape(N_BLOCKS, 1, BLK),
        W1, b1.reshape(EMBED, 1),
        W2, b2.reshape(EMBED, 1),
        Wfc, bfc.reshape(NUM_CLASS, 1),
    )
    return loss_sum[0, 0] / B


# speedup vs baseline: 1.6664x; 1.6664x over previous
import functools

import jax
import jax.numpy as jnp
from jax import lax
from jax.experimental import pallas as pl
from jax.experimental.pallas import tpu as pltpu
from jax.experimental.pallas import tpu_sc as plsc

VOCAB = 1000000
EMBED = 64
NUM_CLASS = 16
B = 16384

BLK = 2048
N_BLOCKS = B // BLK

NBUF = 8


def _sc_gather_cols(tableT, text):
    info = plsc.get_sparse_core_info()
    nc, ns = info.num_cores, info.num_subcores
    nw = nc * ns
    b_per_w = B // nw
    n_outer = b_per_w // NBUF

    mesh = plsc.VectorSubcoreMesh(core_axis_name="c", subcore_axis_name="s")

    @functools.partial(
        pl.kernel,
        mesh=mesh,
        out_type=jax.ShapeDtypeStruct((EMBED, B), jnp.float32),
        compiler_params=pltpu.CompilerParams(needs_layout_passes=False),
        scratch_types=[
            pltpu.VMEM((NBUF * EMBED, 128), jnp.float32),
            pltpu.VMEM((EMBED, b_per_w), jnp.float32),
            pltpu.VMEM((b_per_w,), jnp.int32),
            pltpu.SemaphoreType.DMA((NBUF,)),
        ],
    )
    def gather_k(tableT_hbm, text_hbm, out_hbm, tiles_v, stage_v, idx_v,
                 fsem):
        wid = lax.axis_index("s") * nc + lax.axis_index("c")
        base = wid * b_per_w

        pltpu.sync_copy(text_hbm.at[pl.ds(base, b_per_w)], idx_v)

        def get_idx(j):
            vec = idx_v[pl.ds((j >> 4) * 16, 16)]
            sel = jnp.where(lax.iota(jnp.int32, 16) == (j & 15), vec, 0)
            return jnp.sum(sel)

        def fetch(tile, slot):
            off = pl.multiple_of(tile << 7, 128)
            pltpu.async_copy(
                tableT_hbm.at[:, pl.ds(off, 128)],
                tiles_v.at[pl.ds(slot * EMBED, EMBED)],
                fsem.at[slot],
            )

        t0 = get_idx(0) >> 7
        fetch(t0, 0)
        fc = jnp.int32(1)
        prev_t = t0
        for s in range(1, NBUF):
            t_s = get_idx(s) >> 7
            pred = t_s != prev_t
            slot = fc & (NBUF - 1)

            @pl.when(pred)
            def _prime_fetch():
                fetch(t_s, slot)

            fc = fc + pred.astype(jnp.int32)
            prev_t = t_s

        def body(j, carry):
            cons_prev, fill_prev, cc, fc, cur_slot = carry

            i = get_idx(j)
            tile = i >> 7
            pred = tile != cons_prev
            slot = cc & (NBUF - 1)

            @pl.when(pred)
            def _wait():
                pltpu.make_async_copy(
                    tableT_hbm.at[:, pl.ds(0, 128)],
                    tiles_v.at[pl.ds(slot * EMBED, EMBED)],
                    fsem.at[slot],
                ).wait()

            cur_slot = jnp.where(pred, slot, cur_slot)
            cc = cc + pred.astype(jnp.int32)

            lane = jnp.full((16,), i & 127, jnp.int32)
            col = jnp.full((16,), j, jnp.int32)
            for k in range(EMBED // 16):
                rows = lax.iota(jnp.int32, 16) + (16 * k)
                vals = plsc.load_gather(
                    tiles_v, [rows + cur_slot * EMBED, lane])
                plsc.store_scatter(stage_v, [rows, col], vals)

            jn = jnp.minimum(j + NBUF, b_per_w - 1)
            live = j < b_per_w - NBUF
            t_n = get_idx(jn) >> 7
            pred_n = (t_n != fill_prev) & live
            fslot = fc & (NBUF - 1)

            @pl.when(pred_n)
            def _refill():
                fetch(t_n, fslot)

            fc = fc + pred_n.astype(jnp.int32)
            fill_prev = jnp.where(live, t_n, fill_prev)
            return tile, fill_prev, cc, fc, cur_slot

        lax.fori_loop(
            0, b_per_w, body,
            (jnp.int32(-1), prev_t, jnp.int32(0), fc, jnp.int32(0)),
            unroll=False)

        pltpu.sync_copy(stage_v, out_hbm.at[:, pl.ds(base, b_per_w)])

    return gather_k(tableT, text)


def _mlp_loss_body(emb_ref, lab_ref, w1_ref, b1_ref, w2_ref, b2_ref,
                   wfc_ref, bfc_ref, out_ref):
    x = emb_ref[...]
    cdims = (((0,), (0,)), ((), ()))
    h = jnp.maximum(
        lax.dot_general(w1_ref[...], x, cdims,
                        preferred_element_type=jnp.float32) + b1_ref[...], 0.0)
    h = jnp.maximum(
        lax.dot_general(w2_ref[...], h, cdims,
                        preferred_element_type=jnp.float32) + b2_ref[...], 0.0)
    logits = lax.dot_general(wfc_ref[...], h, cdims,
                             preferred_element_type=jnp.float32) + bfc_ref[...]
    m = jnp.max(logits, axis=0, keepdims=True)
    lse = jnp.log(jnp.sum(jnp.exp(logits - m), axis=0, keepdims=True)) + m
    lab = lab_ref[...].reshape(1, BLK)
    cls = lax.broadcasted_iota(jnp.int32, (NUM_CLASS, BLK), 0)
    picked = jnp.sum(jnp.where(cls == lab, logits, 0.0),
                     axis=0, keepdims=True)
    part = jnp.sum(lse - picked, axis=(0, 1), keepdims=True)

    @pl.when(pl.program_id(0) == 0)
    def _init():
        out_ref[...] = jnp.zeros_like(out_ref)

    out_ref[...] += part


def _mlp_loss(embT, labels3d, W1, b1, W2, b2, Wfc, bfc):
    return pl.pallas_call(
        _mlp_loss_body,
        grid=(N_BLOCKS,),
        in_specs=[
            pl.BlockSpec((EMBED, BLK), lambda i: (0, i)),
            pl.BlockSpec((1, 1, BLK), lambda i: (i, 0, 0)),
            pl.BlockSpec((EMBED, EMBED), lambda i: (0, 0)),
            pl.BlockSpec((EMBED, 1), lambda i: (0, 0)),
            pl.BlockSpec((EMBED, EMBED), lambda i: (0, 0)),
            pl.BlockSpec((EMBED, 1), lambda i: (0, 0)),
            pl.BlockSpec((EMBED, NUM_CLASS), lambda i: (0, 0)),
            pl.BlockSpec((NUM_CLASS, 1), lambda i: (0, 0)),
        ],
        out_specs=pl.BlockSpec((1, 1), lambda i: (0, 0)),
        out_shape=jax.ShapeDtypeStruct((1, 1), jnp.float32),
    )(embT, labels3d, W1, b1, W2, b2, Wfc, bfc)


def kernel(text, offsets, labels, table, W1, b1, W2, b2, Wfc, bfc):
    text_s, labels_s = lax.sort(
        (text.astype(jnp.int32), labels.astype(jnp.int32)), num_keys=1)
    embT = _sc_gather_cols(table.T, text_s)
    loss_sum = _mlp_loss(
        embT,
        labels_s.reshape(N_BLOCKS, 1, BLK),
        W1, b1.reshape(EMBED, 1),
        W2, b2.reshape(EMBED, 1),
        Wfc, bfc.reshape(NUM_CLASS, 1),
    )
    return loss_sum[0, 0] / B


# speedup vs baseline: 1.8609x; 1.1167x over previous
import functools

import jax
import jax.numpy as jnp
from jax import lax
from jax.experimental import pallas as pl
from jax.experimental.pallas import tpu as pltpu
from jax.experimental.pallas import tpu_sc as plsc

VOCAB = 1000000
EMBED = 64
NUM_CLASS = 16
B = 16384

BLK = 2048
N_BLOCKS = B // BLK

NBUF = 8


def _sc_gather_cols(tableT, text):
    info = plsc.get_sparse_core_info()
    nc, ns = info.num_cores, info.num_subcores
    nw = nc * ns
    b_per_w = B // nw
    n_outer = b_per_w // NBUF

    mesh = plsc.VectorSubcoreMesh(core_axis_name="c", subcore_axis_name="s")

    @functools.partial(
        pl.kernel,
        mesh=mesh,
        out_type=jax.ShapeDtypeStruct((EMBED, B), jnp.float32),
        compiler_params=pltpu.CompilerParams(needs_layout_passes=False),
        scratch_types=[
            pltpu.VMEM((NBUF * EMBED, 128), jnp.float32),
            pltpu.VMEM((EMBED, b_per_w), jnp.float32),
            pltpu.VMEM((b_per_w,), jnp.int32),
            pltpu.SemaphoreType.DMA((NBUF,)),
        ],
    )
    def gather_k(tableT_hbm, text_hbm, out_hbm, tiles_v, stage_v, idx_v,
                 fsem):
        wid = lax.axis_index("s") * nc + lax.axis_index("c")
        base = wid * b_per_w

        pltpu.sync_copy(text_hbm.at[pl.ds(base, b_per_w)], idx_v)

        def get_idx(j):
            vec = idx_v[pl.ds((j >> 4) * 16, 16)]
            sel = jnp.where(lax.iota(jnp.int32, 16) == (j & 15), vec, 0)
            return jnp.sum(sel)

        def fetch(tile, slot):
            off = pl.multiple_of(tile << 7, 128)
            pltpu.async_copy(
                tableT_hbm.at[:, pl.ds(off, 128)],
                tiles_v.at[pl.ds(slot * EMBED, EMBED)],
                fsem.at[slot],
            )

        t0 = get_idx(0) >> 7
        fetch(t0, 0)
        fc = jnp.int32(1)
        prev_t = t0
        for s in range(1, NBUF):
            t_s = get_idx(s) >> 7
            pred = t_s != prev_t
            slot = fc & (NBUF - 1)

            @pl.when(pred)
            def _prime_fetch():
                fetch(t_s, slot)

            fc = fc + pred.astype(jnp.int32)
            prev_t = t_s

        def body(j, carry):
            cons_prev, fill_prev, cc, fc, cur_slot, jf = carry

            for _ in range(2):
                t_n = get_idx(jnp.minimum(jf, b_per_w - 1)) >> 7
                adv = (jf < b_per_w) & (fc - cc < NBUF - 1)
                pred_n = (t_n != fill_prev) & adv
                fslot = fc & (NBUF - 1)

                @pl.when(pred_n)
                def _refill():
                    fetch(t_n, fslot)

                fc = fc + pred_n.astype(jnp.int32)
                fill_prev = jnp.where(adv, t_n, fill_prev)
                jf = jf + adv.astype(jnp.int32)

            i = get_idx(j)
            tile = i >> 7
            pred = tile != cons_prev
            slot = cc & (NBUF - 1)

            @pl.when(pred)
            def _wait():
                pltpu.make_async_copy(
                    tableT_hbm.at[:, pl.ds(0, 128)],
                    tiles_v.at[pl.ds(slot * EMBED, EMBED)],
                    fsem.at[slot],
                ).wait()

            cur_slot = jnp.where(pred, slot, cur_slot)
            cc = cc + pred.astype(jnp.int32)

            lane = jnp.full((16,), i & 127, jnp.int32)
            col = jnp.full((16,), j, jnp.int32)
            for k in range(EMBED // 16):
                rows = lax.iota(jnp.int32, 16) + (16 * k)
                vals = plsc.load_gather(
                    tiles_v, [rows + cur_slot * EMBED, lane])
                plsc.store_scatter(stage_v, [rows, col], vals)

            return tile, fill_prev, cc, fc, cur_slot, jf

        lax.fori_loop(
            0, b_per_w, body,
            (jnp.int32(-1), prev_t, jnp.int32(0), fc, jnp.int32(0),
             jnp.int32(NBUF)),
            unroll=False)

        pltpu.sync_copy(stage_v, out_hbm.at[:, pl.ds(base, b_per_w)])

    return gather_k(tableT, text)


def _mlp_loss_body(emb_ref, lab_ref, w1_ref, b1_ref, w2_ref, b2_ref,
                   wfc_ref, bfc_ref, out_ref):
    x = emb_ref[...]
    cdims = (((0,), (0,)), ((), ()))
    h = jnp.maximum(
        lax.dot_general(w1_ref[...], x, cdims,
                        preferred_element_type=jnp.float32) + b1_ref[...], 0.0)
    h = jnp.maximum(
        lax.dot_general(w2_ref[...], h, cdims,
                        preferred_element_type=jnp.float32) + b2_ref[...], 0.0)
    logits = lax.dot_general(wfc_ref[...], h, cdims,
                             preferred_element_type=jnp.float32) + bfc_ref[...]
    m = jnp.max(logits, axis=0, keepdims=True)
    lse = jnp.log(jnp.sum(jnp.exp(logits - m), axis=0, keepdims=True)) + m
    lab = lab_ref[...].reshape(1, BLK)
    cls = lax.broadcasted_iota(jnp.int32, (NUM_CLASS, BLK), 0)
    picked = jnp.sum(jnp.where(cls == lab, logits, 0.0),
                     axis=0, keepdims=True)
    part = jnp.sum(lse - picked, axis=(0, 1), keepdims=True)

    @pl.when(pl.program_id(0) == 0)
    def _init():
        out_ref[...] = jnp.zeros_like(out_ref)

    out_ref[...] += part


def _mlp_loss(embT, labels3d, W1, b1, W2, b2, Wfc, bfc):
    return pl.pallas_call(
        _mlp_loss_body,
        grid=(N_BLOCKS,),
        in_specs=[
            pl.BlockSpec((EMBED, BLK), lambda i: (0, i)),
            pl.BlockSpec((1, 1, BLK), lambda i: (i, 0, 0)),
            pl.BlockSpec((EMBED, EMBED), lambda i: (0, 0)),
            pl.BlockSpec((EMBED, 1), lambda i: (0, 0)),
            pl.BlockSpec((EMBED, EMBED), lambda i: (0, 0)),
            pl.BlockSpec((EMBED, 1), lambda i: (0, 0)),
            pl.BlockSpec((EMBED, NUM_CLASS), lambda i: (0, 0)),
            pl.BlockSpec((NUM_CLASS, 1), lambda i: (0, 0)),
        ],
        out_specs=pl.BlockSpec((1, 1), lambda i: (0, 0)),
        out_shape=jax.ShapeDtypeStruct((1, 1), jnp.float32),
    )(embT, labels3d, W1, b1, W2, b2, Wfc, bfc)


def kernel(text, offsets, labels, table, W1, b1, W2, b2, Wfc, bfc):
    text_s, labels_s = lax.sort(
        (text.astype(jnp.int32), labels.astype(jnp.int32)), num_keys=1)
    embT = _sc_gather_cols(table.T, text_s)
    loss_sum = _mlp_loss(
        embT,
        labels_s.reshape(N_BLOCKS, 1, BLK),
        W1, b1.reshape(EMBED, 1),
        W2, b2.reshape(EMBED, 1),
        Wfc, bfc.reshape(NUM_CLASS, 1),
    )
    return loss_sum[0, 0] / B
